# baseline (speedup 1.0000x reference)
import functools

import jax
import jax.numpy as jnp
from jax import lax
from jax.experimental import pallas as pl
from jax.experimental.pallas import tpu as pltpu
from jax.experimental.pallas import tpu_sc as plsc

B, K, D = 16384, 32, 32
NR = 64
NC, NS, L = 2, 16, 16
NW = NC * NS
BPW = B // NW
C = 16
NCHUNK = BPW // C
G = 128
BBLK = 2048


def _scores_body(u_ref, w_ref, rel_ref, s_ref):
    m = lax.dot_general(rel_ref[...], w_ref[...], (((1,), (0,)), ((), ())),
                        preferred_element_type=jnp.float32,
                        precision=lax.Precision.HIGHEST)
    s = lax.dot_general(u_ref[...], m, (((1,), (1,)), ((), ())),
                        preferred_element_type=jnp.float32,
                        precision=lax.Precision.HIGHEST)
    s_ref[...] = jnp.exp(s - jnp.max(s, axis=1, keepdims=True))


def _scores_tc(user_emb, W, relation_table):
    return pl.pallas_call(
        _scores_body,
        grid=(B // BBLK,),
        in_specs=[
            pl.BlockSpec((BBLK, D), lambda i: (i, 0)),
            pl.BlockSpec((D, D), lambda i: (0, 0)),
            pl.BlockSpec((NR, D), lambda i: (0, 0)),
        ],
        out_specs=pl.BlockSpec((BBLK, NR), lambda i: (i, 0)),
        out_shape=jax.ShapeDtypeStruct((B, NR), jnp.float32),
    )(user_emb, W, relation_table)


TR = 4096
NE = 1000000


def _transpose_body(xt_ref, out_ref):
    y = lax.dot_general(xt_ref[...], jnp.eye(D, dtype=jnp.float32),
                        (((0,), (0,)), ((), ())),
                        preferred_element_type=jnp.float32,
                        precision=lax.Precision.HIGHEST)
    y3 = y.reshape(TR // 4, 4, D)
    for q in range(4):
        out_ref[:, q * D:(q + 1) * D] = y3[:, q, :]


def _table_rowmajor_tc(tabT):
    return pl.pallas_call(
        _transpose_body,
        grid=(pl.cdiv(NE, TR),),
        in_specs=[pl.BlockSpec((D, TR), lambda i: (0, i))],
        out_specs=pl.BlockSpec((TR // 4, 4 * D), lambda i: (i, 0)),
        out_shape=jax.ShapeDtypeStruct((NE // 4, 4 * D), jnp.float32),
    )(tabT)


def _wid():
    return lax.axis_index("s") * NC + lax.axis_index("c")


def _vgather(ref, idx):
    return plsc.load_gather(ref, [idx])


def _gather_rows(tab_hbm, idx_ref, dst_ref, sem):
    return pltpu.async_copy(tab_hbm.at[idx_ref], dst_ref, sem)


def _agg_body(nid_hbm, eid_hbm, rid_hbm, s_hbm, tab_hbm, out_hbm,
              idx_v, rid_v, s_v, eidx_v, rows_v, out_v,
              sem_rows, sem_self):
    w = _wid()
    b0 = w * BPW
    ngc = (C * K) // G

    pltpu.sync_copy(nid_hbm.at[pl.ds(w * (BPW * K // G), BPW * K // G)],
                    idx_v)
    pltpu.sync_copy(rid_hbm.at[pl.ds(b0 * K, BPW * K)], rid_v)
    pltpu.sync_copy(s_hbm.at[pl.ds(b0 * NR, BPW * NR)], s_v)
    pltpu.sync_copy(eid_hbm.at[pl.ds(w * (BPW // G), BPW // G)], eidx_v)

    self_cps = [
        _gather_rows(tab_hbm, eidx_v.at[g],
                     out_v.at[pl.ds(g * G, G)], sem_self)
        for g in range(BPW // G)
    ]
    for cp in self_cps:
        cp.wait()

    def chunk(ci, carry):
        cps = [
            _gather_rows(tab_hbm, idx_v.at[ci * ngc + g],
                         rows_v.at[pl.ds(g * G, G)], sem_rows)
            for g in range(ngc)
        ]
        for cp in cps:
            cp.wait()

        for b in range(C):
            bb = ci * C + b
            a0 = jnp.zeros((L,), jnp.float32)
            a1 = jnp.zeros((L,), jnp.float32)
            tot = jnp.zeros((L,), jnp.float32)
            for k in range(K):
                rk = _vgather(rid_v, jnp.full((L,), bb * K + k, jnp.int32))
                ek = _vgather(s_v, rk + bb * NR)
                tot = tot + ek
                r = b * K + k
                a0 = a0 + ek * rows_v[r, 0:L]
                a1 = a1 + ek * rows_v[r, L:D]
            inv = jnp.full((L,), 1.0, jnp.float32) / tot
            out_v[bb, 0:L] = jnp.maximum(out_v[bb, 0:L] + a0 * inv, 0.0)
            out_v[bb, L:D] = jnp.maximum(out_v[bb, L:D] + a1 * inv, 0.0)
        return carry

    lax.fori_loop(0, NCHUNK, chunk, 0)
    pltpu.sync_copy(out_v, out_hbm.at[pl.ds(b0, BPW)])


@functools.cache
def _agg_sc():
  return pl.kernel(
    _agg_body,
    out_type=jax.ShapeDtypeStruct((B, D), jnp.float32),
    mesh=plsc.VectorSubcoreMesh(core_axis_name="c", subcore_axis_name="s",
                                num_cores=NC, num_subcores=NS),
    compiler_params=pltpu.CompilerParams(needs_layout_passes=False,
                                         use_tc_tiling_on_sc=False),
    scratch_types=[
        pltpu.VMEM((BPW * K // G, G), jnp.int32),
        pltpu.VMEM((BPW * K,), jnp.int32),
        pltpu.VMEM((BPW * NR,), jnp.float32),
        pltpu.VMEM((BPW // G, G), jnp.int32),
        pltpu.VMEM((C * K, D), jnp.float32),
        pltpu.VMEM((BPW, D), jnp.float32),
        pltpu.SemaphoreType.DMA,
        pltpu.SemaphoreType.DMA,
    ],
  )


def kernel(user_emb, entity_ids, neigh_ent_ids, neigh_rel_ids,
           entity_table, relation_table, W):
    s = _scores_tc(user_emb.astype(jnp.float32), W.astype(jnp.float32),
                   relation_table.astype(jnp.float32))
    nid = neigh_ent_ids.astype(jnp.int32).reshape(B * K // G, G)
    rid = neigh_rel_ids.astype(jnp.int32).reshape(B * K)
    eid = entity_ids.astype(jnp.int32).reshape(B // G, G)
    tab_rm = _table_rowmajor_tc(entity_table.astype(jnp.float32).T)
    return _agg_sc()(nid, eid, rid, s.reshape(B * NR),
                     tab_rm.reshape(NE, D))


# speedup vs baseline: 1.5763x; 1.5763x over previous
import functools

import jax
import jax.numpy as jnp
from jax import lax
from jax.experimental import pallas as pl
from jax.experimental.pallas import tpu as pltpu
from jax.experimental.pallas import tpu_sc as plsc

B, K, D = 16384, 32, 32
NR = 64
NC, NS, L = 2, 16, 16
NW = NC * NS
BPW = B // NW
C = 16
NCHUNK = BPW // C
G = 128
BBLK = 2048


def _scores_body(u_ref, w_ref, rel_ref, s_ref):
    m = lax.dot_general(rel_ref[...], w_ref[...], (((1,), (0,)), ((), ())),
                        preferred_element_type=jnp.float32,
                        precision=lax.Precision.HIGHEST)
    s = lax.dot_general(u_ref[...], m, (((1,), (1,)), ((), ())),
                        preferred_element_type=jnp.float32,
                        precision=lax.Precision.HIGHEST)
    s_ref[...] = jnp.exp(s - jnp.max(s, axis=1, keepdims=True))


def _scores_tc(user_emb, W, relation_table):
    return pl.pallas_call(
        _scores_body,
        grid=(B // BBLK,),
        in_specs=[
            pl.BlockSpec((BBLK, D), lambda i: (i, 0)),
            pl.BlockSpec((D, D), lambda i: (0, 0)),
            pl.BlockSpec((NR, D), lambda i: (0, 0)),
        ],
        out_specs=pl.BlockSpec((BBLK, NR), lambda i: (i, 0)),
        out_shape=jax.ShapeDtypeStruct((B, NR), jnp.float32),
    )(user_emb, W, relation_table)


TR = 8192
NE = 1000000


def _transpose_body(xt_ref, out_ref):
    y = jnp.transpose(xt_ref[...])
    y3 = y.reshape(TR // 4, 4, D)
    for q in range(4):
        out_ref[:, q * D:(q + 1) * D] = y3[:, q, :]


def _table_rowmajor_tc(tabT):
    return pl.pallas_call(
        _transpose_body,
        grid=(pl.cdiv(NE, TR),),
        in_specs=[pl.BlockSpec((D, TR), lambda i: (0, i))],
        out_specs=pl.BlockSpec((TR // 4, 4 * D), lambda i: (i, 0)),
        out_shape=jax.ShapeDtypeStruct((NE // 4, 4 * D), jnp.float32),
    )(tabT)


def _wid():
    return lax.axis_index("s") * NC + lax.axis_index("c")


def _vgather(ref, idx):
    return plsc.load_gather(ref, [idx])


def _gather_rows(tab_hbm, idx_ref, dst_ref, sem):
    return pltpu.async_copy(tab_hbm.at[idx_ref], dst_ref, sem)


def _agg_body(nid_hbm, eid_hbm, rid_hbm, s_hbm, tab_hbm, out_hbm,
              idx_v, rid_v, s_v, eidx_v, rows_v, out_v,
              sem_rows, sem_self):
    w = _wid()
    b0 = w * BPW
    ngc = (C * K) // G

    pltpu.sync_copy(nid_hbm.at[pl.ds(w * (BPW * K // G), BPW * K // G)],
                    idx_v)
    pltpu.sync_copy(rid_hbm.at[pl.ds(b0 * K, BPW * K)], rid_v)
    pltpu.sync_copy(s_hbm.at[pl.ds(b0 * NR, BPW * NR)], s_v)
    pltpu.sync_copy(eid_hbm.at[pl.ds(w * (BPW // G), BPW // G)], eidx_v)

    self_cps = [
        _gather_rows(tab_hbm, eidx_v.at[g],
                     out_v.at[pl.ds(g * G, G)], sem_self)
        for g in range(BPW // G)
    ]
    for cp in self_cps:
        cp.wait()

    def chunk(ci, carry):
        cps = [
            _gather_rows(tab_hbm, idx_v.at[ci * ngc + g],
                         rows_v.at[pl.ds(g * G, G)], sem_rows)
            for g in range(ngc)
        ]
        for cp in cps:
            cp.wait()

        for b in range(C):
            bb = ci * C + b
            a0 = jnp.zeros((L,), jnp.float32)
            a1 = jnp.zeros((L,), jnp.float32)
            tot = jnp.zeros((L,), jnp.float32)
            for k in range(K):
                rk = _vgather(rid_v, jnp.full((L,), bb * K + k, jnp.int32))
                ek = _vgather(s_v, rk + bb * NR)
                tot = tot + ek
                r = b * K + k
                a0 = a0 + ek * rows_v[r, 0:L]
                a1 = a1 + ek * rows_v[r, L:D]
            inv = jnp.full((L,), 1.0, jnp.float32) / tot
            out_v[bb, 0:L] = jnp.maximum(out_v[bb, 0:L] + a0 * inv, 0.0)
            out_v[bb, L:D] = jnp.maximum(out_v[bb, L:D] + a1 * inv, 0.0)
        return carry

    lax.fori_loop(0, NCHUNK, chunk, 0)
    pltpu.sync_copy(out_v, out_hbm.at[pl.ds(b0, BPW)])


@functools.cache
def _agg_sc():
  return pl.kernel(
    _agg_body,
    out_type=jax.ShapeDtypeStruct((B, D), jnp.float32),
    mesh=plsc.VectorSubcoreMesh(core_axis_name="c", subcore_axis_name="s",
                                num_cores=NC, num_subcores=NS),
    compiler_params=pltpu.CompilerParams(needs_layout_passes=False,
                                         use_tc_tiling_on_sc=False),
    scratch_types=[
        pltpu.VMEM((BPW * K // G, G), jnp.int32),
        pltpu.VMEM((BPW * K,), jnp.int32),
        pltpu.VMEM((BPW * NR,), jnp.float32),
        pltpu.VMEM((BPW // G, G), jnp.int32),
        pltpu.VMEM((C * K, D), jnp.float32),
        pltpu.VMEM((BPW, D), jnp.float32),
        pltpu.SemaphoreType.DMA,
        pltpu.SemaphoreType.DMA,
    ],
  )


def kernel(user_emb, entity_ids, neigh_ent_ids, neigh_rel_ids,
           entity_table, relation_table, W):
    s = _scores_tc(user_emb.astype(jnp.float32), W.astype(jnp.float32),
                   relation_table.astype(jnp.float32))
    nid = neigh_ent_ids.astype(jnp.int32).reshape(B * K // G, G)
    rid = neigh_rel_ids.astype(jnp.int32).reshape(B * K)
    eid = entity_ids.astype(jnp.int32).reshape(B // G, G)
    tab_rm = _table_rowmajor_tc(entity_table.astype(jnp.float32).T)
    return _agg_sc()(nid, eid, rid, s.reshape(B * NR),
                     tab_rm.reshape(NE, D))


# speedup vs baseline: 2.7416x; 1.7392x over previous
import functools

import jax
import jax.numpy as jnp
from jax import lax
from jax.experimental import pallas as pl
from jax.experimental.pallas import tpu as pltpu
from jax.experimental.pallas import tpu_sc as plsc

B, K, D = 16384, 32, 32
NR = 64
NC, NS, L = 2, 16, 16
NW = NC * NS
BPW = B // NW
C = 16
NCHUNK = BPW // C
G = 128
BBLK = 2048


def _scores_body(u_ref, w_ref, rel_ref, s_ref):
    m = lax.dot_general(rel_ref[...], w_ref[...], (((1,), (0,)), ((), ())),
                        preferred_element_type=jnp.float32,
                        precision=lax.Precision.HIGHEST)
    s = lax.dot_general(u_ref[...], m, (((1,), (1,)), ((), ())),
                        preferred_element_type=jnp.float32,
                        precision=lax.Precision.HIGHEST)
    s_ref[...] = jnp.exp(s - jnp.max(s, axis=1, keepdims=True))


def _scores_tc(user_emb, W, relation_table):
    return pl.pallas_call(
        _scores_body,
        grid=(B // BBLK,),
        in_specs=[
            pl.BlockSpec((BBLK, D), lambda i: (i, 0)),
            pl.BlockSpec((D, D), lambda i: (0, 0)),
            pl.BlockSpec((NR, D), lambda i: (0, 0)),
        ],
        out_specs=pl.BlockSpec((BBLK, NR), lambda i: (i, 0)),
        out_shape=jax.ShapeDtypeStruct((B, NR), jnp.float32),
    )(user_emb, W, relation_table)


TR = 16384
TR4 = TR // 4
NE = 1000000
NTB = (NE + TR - 1) // TR
NEP = NTB * TR


def _transpose_body(xt_ref, out_ref):
    x = xt_ref[...]
    s = jnp.concatenate([x[:, q * TR4:(q + 1) * TR4] for q in range(4)],
                        axis=0)
    out_ref[...] = jnp.transpose(s)


def _table_rowmajor_tc(tabT):
    return pl.pallas_call(
        _transpose_body,
        grid=(NTB,),
        in_specs=[pl.BlockSpec((D, TR), lambda i: (0, i))],
        out_specs=pl.BlockSpec((TR4, 4 * D), lambda i: (i, 0)),
        out_shape=jax.ShapeDtypeStruct((NEP // 4, 4 * D), jnp.float32),
    )(tabT)


def _sigma(r):
    loc = r % TR
    return (r - loc) + 4 * (loc % TR4) + loc // TR4


def _wid():
    return lax.axis_index("s") * NC + lax.axis_index("c")


def _vgather(ref, idx):
    return plsc.load_gather(ref, [idx])


def _gather_rows(tab_hbm, idx_ref, dst_ref, sem):
    return pltpu.async_copy(tab_hbm.at[idx_ref], dst_ref, sem)


def _agg_body(nid_hbm, eid_hbm, rid_hbm, s_hbm, tab_hbm, out_hbm,
              idx_v, rid_v, s_v, eidx_v, rows_v, out_v,
              sem_rows, sem_self):
    w = _wid()
    b0 = w * BPW
    ngc = (C * K) // G

    pltpu.sync_copy(nid_hbm.at[pl.ds(w * (BPW * K // G), BPW * K // G)],
                    idx_v)
    pltpu.sync_copy(rid_hbm.at[pl.ds(b0 * K, BPW * K)], rid_v)
    pltpu.sync_copy(s_hbm.at[pl.ds(b0 * NR, BPW * NR)], s_v)
    pltpu.sync_copy(eid_hbm.at[pl.ds(w * (BPW // G), BPW // G)], eidx_v)

    self_cps = [
        _gather_rows(tab_hbm, eidx_v.at[g],
                     out_v.at[pl.ds(g * G, G)], sem_self)
        for g in range(BPW // G)
    ]
    for cp in self_cps:
        cp.wait()

    def chunk(ci, carry):
        cps = [
            _gather_rows(tab_hbm, idx_v.at[ci * ngc + g],
                         rows_v.at[pl.ds(g * G, G)], sem_rows)
            for g in range(ngc)
        ]
        for cp in cps:
            cp.wait()

        for b in range(C):
            bb = ci * C + b
            a0 = jnp.zeros((L,), jnp.float32)
            a1 = jnp.zeros((L,), jnp.float32)
            tot = jnp.zeros((L,), jnp.float32)
            for k in range(K):
                rk = _vgather(rid_v, jnp.full((L,), bb * K + k, jnp.int32))
                ek = _vgather(s_v, rk + bb * NR)
                tot = tot + ek
                r = b * K + k
                a0 = a0 + ek * rows_v[r, 0:L]
                a1 = a1 + ek * rows_v[r, L:D]
            inv = jnp.full((L,), 1.0, jnp.float32) / tot
            out_v[bb, 0:L] = jnp.maximum(out_v[bb, 0:L] + a0 * inv, 0.0)
            out_v[bb, L:D] = jnp.maximum(out_v[bb, L:D] + a1 * inv, 0.0)
        return carry

    lax.fori_loop(0, NCHUNK, chunk, 0)
    pltpu.sync_copy(out_v, out_hbm.at[pl.ds(b0, BPW)])


@functools.cache
def _agg_sc():
  return pl.kernel(
    _agg_body,
    out_type=jax.ShapeDtypeStruct((B, D), jnp.float32),
    mesh=plsc.VectorSubcoreMesh(core_axis_name="c", subcore_axis_name="s",
                                num_cores=NC, num_subcores=NS),
    compiler_params=pltpu.CompilerParams(needs_layout_passes=False,
                                         use_tc_tiling_on_sc=False),
    scratch_types=[
        pltpu.VMEM((BPW * K // G, G), jnp.int32),
        pltpu.VMEM((BPW * K,), jnp.int32),
        pltpu.VMEM((BPW * NR,), jnp.float32),
        pltpu.VMEM((BPW // G, G), jnp.int32),
        pltpu.VMEM((C * K, D), jnp.float32),
        pltpu.VMEM((BPW, D), jnp.float32),
        pltpu.SemaphoreType.DMA,
        pltpu.SemaphoreType.DMA,
    ],
  )


def kernel(user_emb, entity_ids, neigh_ent_ids, neigh_rel_ids,
           entity_table, relation_table, W):
    s = _scores_tc(user_emb.astype(jnp.float32), W.astype(jnp.float32),
                   relation_table.astype(jnp.float32))
    nid = _sigma(neigh_ent_ids.astype(jnp.int32)).reshape(B * K // G, G)
    rid = neigh_rel_ids.astype(jnp.int32).reshape(B * K)
    eid = _sigma(entity_ids.astype(jnp.int32)).reshape(B // G, G)
    tab_rm = _table_rowmajor_tc(entity_table.astype(jnp.float32).T)
    return _agg_sc()(nid, eid, rid, s.reshape(B * NR),
                     tab_rm.reshape(NEP, D))


# speedup vs baseline: 2.8953x; 1.0561x over previous
import functools

import jax
import jax.numpy as jnp
from jax import lax
from jax.experimental import pallas as pl
from jax.experimental.pallas import tpu as pltpu
from jax.experimental.pallas import tpu_sc as plsc

B, K, D = 16384, 32, 32
NR = 64
NC, NS, L = 2, 16, 16
NW = NC * NS
BPW = B // NW
C = 16
NCHUNK = BPW // C
G = 128
BBLK = 2048


def _scores_body(u_ref, w_ref, rel_ref, s_ref):
    m = lax.dot_general(rel_ref[...], w_ref[...], (((1,), (0,)), ((), ())),
                        preferred_element_type=jnp.float32,
                        precision=lax.Precision.HIGHEST)
    s = lax.dot_general(u_ref[...], m, (((1,), (1,)), ((), ())),
                        preferred_element_type=jnp.float32,
                        precision=lax.Precision.HIGHEST)
    s_ref[...] = jnp.exp(s - jnp.max(s, axis=1, keepdims=True))


def _scores_tc(user_emb, W, relation_table):
    return pl.pallas_call(
        _scores_body,
        grid=(B // BBLK,),
        in_specs=[
            pl.BlockSpec((BBLK, D), lambda i: (i, 0)),
            pl.BlockSpec((D, D), lambda i: (0, 0)),
            pl.BlockSpec((NR, D), lambda i: (0, 0)),
        ],
        out_specs=pl.BlockSpec((BBLK, NR), lambda i: (i, 0)),
        out_shape=jax.ShapeDtypeStruct((B, NR), jnp.float32),
    )(user_emb, W, relation_table)


TR = 16384
TR4 = TR // 4
NE = 1000000
NTB = (NE + TR - 1) // TR
NEP = NTB * TR


def _transpose_body(xt_ref, out_ref):
    x = xt_ref[...]
    s = jnp.concatenate([x[:, q * TR4:(q + 1) * TR4] for q in range(4)],
                        axis=0)
    out_ref[...] = jnp.transpose(s)


def _table_rowmajor_tc(tabT):
    return pl.pallas_call(
        _transpose_body,
        grid=(NTB,),
        in_specs=[pl.BlockSpec((D, TR), lambda i: (0, i))],
        out_specs=pl.BlockSpec((TR4, 4 * D), lambda i: (i, 0)),
        out_shape=jax.ShapeDtypeStruct((NEP // 4, 4 * D), jnp.float32),
    )(tabT)


def _sigma(r):
    loc = r % TR
    return (r - loc) + 4 * (loc % TR4) + loc // TR4


def _wid():
    return lax.axis_index("s") * NC + lax.axis_index("c")


def _vgather(ref, idx):
    return plsc.load_gather(ref, [idx])


def _gather_rows(tab_hbm, idx_ref, dst_ref, sem):
    return pltpu.async_copy(tab_hbm.at[idx_ref], dst_ref, sem)


def _agg_body(nid_hbm, eid_hbm, rid_hbm, s_hbm, tab_hbm, out_hbm,
              idx_v, rid_v, s_v, eidx_v, rows_a, rows_b, out_v,
              sem_a, sem_b, sem_self):
    w = _wid()
    b0 = w * BPW
    ngc = (C * K) // G

    pltpu.sync_copy(nid_hbm.at[pl.ds(w * (BPW * K // G), BPW * K // G)],
                    idx_v)
    pltpu.sync_copy(rid_hbm.at[pl.ds(b0 * K, BPW * K)], rid_v)
    pltpu.sync_copy(s_hbm.at[pl.ds(b0 * NR, BPW * NR)], s_v)
    pltpu.sync_copy(eid_hbm.at[pl.ds(w * (BPW // G), BPW // G)], eidx_v)

    self_cps = [
        _gather_rows(tab_hbm, eidx_v.at[g],
                     out_v.at[pl.ds(g * G, G)], sem_self)
        for g in range(BPW // G)
    ]
    for cp in self_cps:
        cp.wait()

    def fire(ci, dst, sem):
        for g in range(ngc):
            _gather_rows(tab_hbm, idx_v.at[ci * ngc + g],
                         dst.at[pl.ds(g * G, G)], sem)

    def drain(dst, sem):
        pltpu.make_async_copy(tab_hbm.at[pl.ds(0, C * K)], dst, sem).wait()

    def compute(ci, rows_v):
        for b in range(C):
            bb = ci * C + b
            a0 = jnp.zeros((L,), jnp.float32)
            a1 = jnp.zeros((L,), jnp.float32)
            tot = jnp.zeros((L,), jnp.float32)
            for k in range(K):
                rk = _vgather(rid_v, jnp.full((L,), bb * K + k, jnp.int32))
                ek = _vgather(s_v, rk + bb * NR)
                tot = tot + ek
                r = b * K + k
                a0 = a0 + ek * rows_v[r, 0:L]
                a1 = a1 + ek * rows_v[r, L:D]
            inv = jnp.full((L,), 1.0, jnp.float32) / tot
            out_v[bb, 0:L] = jnp.maximum(out_v[bb, 0:L] + a0 * inv, 0.0)
            out_v[bb, L:D] = jnp.maximum(out_v[bb, L:D] + a1 * inv, 0.0)

    fire(0, rows_a, sem_a)

    def pair(g2, carry):
        ci = g2 * 2
        fire(ci + 1, rows_b, sem_b)
        drain(rows_a, sem_a)
        compute(ci, rows_a)

        @pl.when(ci + 2 < NCHUNK)
        def _():
            fire(ci + 2, rows_a, sem_a)
        drain(rows_b, sem_b)
        compute(ci + 1, rows_b)
        return carry

    lax.fori_loop(0, NCHUNK // 2, pair, 0)
    pltpu.sync_copy(out_v, out_hbm.at[pl.ds(b0, BPW)])


@functools.cache
def _agg_sc():
  return pl.kernel(
    _agg_body,
    out_type=jax.ShapeDtypeStruct((B, D), jnp.float32),
    mesh=plsc.VectorSubcoreMesh(core_axis_name="c", subcore_axis_name="s",
                                num_cores=NC, num_subcores=NS),
    compiler_params=pltpu.CompilerParams(needs_layout_passes=False,
                                         use_tc_tiling_on_sc=False),
    scratch_types=[
        pltpu.VMEM((BPW * K // G, G), jnp.int32),
        pltpu.VMEM((BPW * K,), jnp.int32),
        pltpu.VMEM((BPW * NR,), jnp.float32),
        pltpu.VMEM((BPW // G, G), jnp.int32),
        pltpu.VMEM((C * K, D), jnp.float32),
        pltpu.VMEM((C * K, D), jnp.float32),
        pltpu.VMEM((BPW, D), jnp.float32),
        pltpu.SemaphoreType.DMA,
        pltpu.SemaphoreType.DMA,
        pltpu.SemaphoreType.DMA,
    ],
  )


def kernel(user_emb, entity_ids, neigh_ent_ids, neigh_rel_ids,
           entity_table, relation_table, W):
    s = _scores_tc(user_emb.astype(jnp.float32), W.astype(jnp.float32),
                   relation_table.astype(jnp.float32))
    nid = _sigma(neigh_ent_ids.astype(jnp.int32)).reshape(B * K // G, G)
    rid = neigh_rel_ids.astype(jnp.int32).reshape(B * K)
    eid = _sigma(entity_ids.astype(jnp.int32)).reshape(B // G, G)
    tab_rm = _table_rowmajor_tc(entity_table.astype(jnp.float32).T)
    return _agg_sc()(nid, eid, rid, s.reshape(B * NR),
                     tab_rm.reshape(NEP, D))


# speedup vs baseline: 3.1317x; 1.0817x over previous
import functools

import jax
import jax.numpy as jnp
from jax import lax
from jax.experimental import pallas as pl
from jax.experimental.pallas import tpu as pltpu
from jax.experimental.pallas import tpu_sc as plsc

B, K, D = 16384, 32, 32
NR = 64
NC, NS, L = 2, 16, 16
NW = NC * NS
BPW = B // NW
C = 16
NCHUNK = BPW // C
G = 128
BBLK = 2048


def _scores_body(u_ref, w_ref, rel_ref, s_ref):
    m = lax.dot_general(rel_ref[...], w_ref[...], (((1,), (0,)), ((), ())),
                        preferred_element_type=jnp.float32,
                        precision=lax.Precision.HIGHEST)
    s = lax.dot_general(u_ref[...], m, (((1,), (1,)), ((), ())),
                        preferred_element_type=jnp.float32,
                        precision=lax.Precision.HIGHEST)
    s_ref[...] = jnp.exp(s - jnp.max(s, axis=1, keepdims=True))


def _scores_tc(user_emb, W, relation_table):
    return pl.pallas_call(
        _scores_body,
        grid=(B // BBLK,),
        in_specs=[
            pl.BlockSpec((BBLK, D), lambda i: (i, 0)),
            pl.BlockSpec((D, D), lambda i: (0, 0)),
            pl.BlockSpec((NR, D), lambda i: (0, 0)),
        ],
        out_specs=pl.BlockSpec((BBLK, NR), lambda i: (i, 0)),
        out_shape=jax.ShapeDtypeStruct((B, NR), jnp.float32),
    )(user_emb, W, relation_table)


TR = 16384
TR4 = TR // 4
NE = 1000000
NTB = (NE + TR - 1) // TR
NEP = NTB * TR


def _transpose_body(xt_ref, out_ref):
    x = xt_ref[...]
    s = jnp.concatenate([x[:, q * TR4:(q + 1) * TR4] for q in range(4)],
                        axis=0)
    out_ref[...] = jnp.transpose(s)


def _table_rowmajor_tc(tabT):
    return pl.pallas_call(
        _transpose_body,
        grid=(NTB,),
        in_specs=[pl.BlockSpec((D, TR), lambda i: (0, i))],
        out_specs=pl.BlockSpec((TR4, 4 * D), lambda i: (i, 0)),
        out_shape=jax.ShapeDtypeStruct((NEP // 4, 4 * D), jnp.float32),
    )(tabT)


def _sigma(r):
    loc = r % TR
    return (r - loc) + 4 * (loc % TR4) + loc // TR4


def _wid():
    return lax.axis_index("s") * NC + lax.axis_index("c")


def _vgather(ref, idx):
    return plsc.load_gather(ref, [idx])


def _gather_rows(tab_hbm, idx_ref, dst_ref, sem):
    return pltpu.async_copy(tab_hbm.at[idx_ref], dst_ref, sem)


def _agg_body(nid_hbm, eid_hbm, w_hbm, tab_hbm, out_hbm,
              idx_v, w_v, eidx_v, rows_a, rows_b, out_v,
              sem_a, sem_b, sem_self):
    w = _wid()
    b0 = w * BPW
    ngc = (C * K) // G

    pltpu.sync_copy(nid_hbm.at[pl.ds(w * (BPW * K // G), BPW * K // G)],
                    idx_v)
    pltpu.sync_copy(w_hbm.at[pl.ds(b0 * K, BPW * K)], w_v)
    pltpu.sync_copy(eid_hbm.at[pl.ds(w * (BPW // G), BPW // G)], eidx_v)

    self_cps = [
        _gather_rows(tab_hbm, eidx_v.at[g],
                     out_v.at[pl.ds(g * G, G)], sem_self)
        for g in range(BPW // G)
    ]
    for cp in self_cps:
        cp.wait()

    def fire(ci, dst, sem):
        for g in range(ngc):
            _gather_rows(tab_hbm, idx_v.at[ci * ngc + g],
                         dst.at[pl.ds(g * G, G)], sem)

    def drain(dst, sem):
        pltpu.make_async_copy(tab_hbm.at[pl.ds(0, C * K)], dst, sem).wait()

    def compute(ci, rows_v):
        for b in range(C):
            bb = ci * C + b
            a0 = jnp.zeros((L,), jnp.float32)
            a1 = jnp.zeros((L,), jnp.float32)
            tot = jnp.zeros((L,), jnp.float32)
            for k in range(K):
                ek = _vgather(w_v, jnp.full((L,), bb * K + k, jnp.int32))
                tot = tot + ek
                r = b * K + k
                a0 = a0 + ek * rows_v[r, 0:L]
                a1 = a1 + ek * rows_v[r, L:D]
            inv = jnp.full((L,), 1.0, jnp.float32) / tot
            out_v[bb, 0:L] = jnp.maximum(out_v[bb, 0:L] + a0 * inv, 0.0)
            out_v[bb, L:D] = jnp.maximum(out_v[bb, L:D] + a1 * inv, 0.0)

    fire(0, rows_a, sem_a)

    def pair(g2, carry):
        ci = g2 * 2
        fire(ci + 1, rows_b, sem_b)
        drain(rows_a, sem_a)
        compute(ci, rows_a)

        @pl.when(ci + 2 < NCHUNK)
        def _():
            fire(ci + 2, rows_a, sem_a)
        drain(rows_b, sem_b)
        compute(ci + 1, rows_b)
        return carry

    lax.fori_loop(0, NCHUNK // 2, pair, 0)
    pltpu.sync_copy(out_v, out_hbm.at[pl.ds(b0, BPW)])


@functools.cache
def _agg_sc():
  return pl.kernel(
    _agg_body,
    out_type=jax.ShapeDtypeStruct((B, D), jnp.float32),
    mesh=plsc.VectorSubcoreMesh(core_axis_name="c", subcore_axis_name="s",
                                num_cores=NC, num_subcores=NS),
    compiler_params=pltpu.CompilerParams(needs_layout_passes=False,
                                         use_tc_tiling_on_sc=False),
    scratch_types=[
        pltpu.VMEM((BPW * K // G, G), jnp.int32),
        pltpu.VMEM((BPW * K,), jnp.float32),
        pltpu.VMEM((BPW // G, G), jnp.int32),
        pltpu.VMEM((C * K, D), jnp.float32),
        pltpu.VMEM((C * K, D), jnp.float32),
        pltpu.VMEM((BPW, D), jnp.float32),
        pltpu.SemaphoreType.DMA,
        pltpu.SemaphoreType.DMA,
        pltpu.SemaphoreType.DMA,
    ],
  )


def kernel(user_emb, entity_ids, neigh_ent_ids, neigh_rel_ids,
           entity_table, relation_table, W):
    s = _scores_tc(user_emb.astype(jnp.float32), W.astype(jnp.float32),
                   relation_table.astype(jnp.float32))
    nid = _sigma(neigh_ent_ids.astype(jnp.int32)).reshape(B * K // G, G)
    eid = _sigma(entity_ids.astype(jnp.int32)).reshape(B // G, G)
    e = jnp.take_along_axis(s, neigh_rel_ids.astype(jnp.int32), axis=1,
                            mode="clip")
    tab_rm = _table_rowmajor_tc(entity_table.astype(jnp.float32).T)
    return _agg_sc()(nid, eid, e.reshape(B * K),
                     tab_rm.reshape(NEP, D))


# speedup vs baseline: 3.2717x; 1.0447x over previous
import functools

import jax
import jax.numpy as jnp
from jax import lax
from jax.experimental import pallas as pl
from jax.experimental.pallas import tpu as pltpu
from jax.experimental.pallas import tpu_sc as plsc

B, K, D = 16384, 32, 32
NR = 64
NC, NS, L = 2, 16, 16
NW = NC * NS
BPW = B // NW
C = 16
NCHUNK = BPW // C
G = 128
BBLK = 4096


def _scores_body(u_ref, w_ref, rel_ref, s_ref):
    m = lax.dot_general(rel_ref[...], w_ref[...], (((1,), (0,)), ((), ())),
                        preferred_element_type=jnp.float32,
                        precision=lax.Precision.HIGHEST)
    s = lax.dot_general(u_ref[...], m, (((1,), (1,)), ((), ())),
                        preferred_element_type=jnp.float32,
                        precision=lax.Precision.HIGHEST)
    s_ref[...] = jnp.exp(s - jnp.max(s, axis=1, keepdims=True))


def _scores_tc(user_emb, W, relation_table):
    return pl.pallas_call(
        _scores_body,
        grid=(B // BBLK,),
        in_specs=[
            pl.BlockSpec((BBLK, D), lambda i: (i, 0)),
            pl.BlockSpec((D, D), lambda i: (0, 0)),
            pl.BlockSpec((NR, D), lambda i: (0, 0)),
        ],
        out_specs=pl.BlockSpec((BBLK, NR), lambda i: (i, 0)),
        out_shape=jax.ShapeDtypeStruct((B, NR), jnp.float32),
    )(user_emb, W, relation_table)


TR = 32768
TR4 = TR // 4
NE = 1000000
NTB = (NE + TR - 1) // TR
NEP = NTB * TR


def _transpose_body(xt_ref, out_ref):
    x = xt_ref[...]
    s = jnp.concatenate([x[:, q * TR4:(q + 1) * TR4] for q in range(4)],
                        axis=0)
    out_ref[...] = jnp.transpose(s)


def _table_rowmajor_tc(tabT):
    return pl.pallas_call(
        _transpose_body,
        grid=(NTB,),
        in_specs=[pl.BlockSpec((D, TR), lambda i: (0, i))],
        out_specs=pl.BlockSpec((TR4, 4 * D), lambda i: (i, 0)),
        out_shape=jax.ShapeDtypeStruct((NEP // 4, 4 * D), jnp.float32),
    )(tabT)


def _sigma(r):
    loc = r % TR
    return (r - loc) + 4 * (loc % TR4) + loc // TR4


def _wid():
    return lax.axis_index("s") * NC + lax.axis_index("c")


def _vgather(ref, idx):
    return plsc.load_gather(ref, [idx])


def _gather_rows(tab_hbm, idx_ref, dst_ref, sem):
    return pltpu.async_copy(tab_hbm.at[idx_ref], dst_ref, sem)


def _agg_body(nid_hbm, eid_hbm, w_hbm, tab_hbm, out_hbm,
              idx_v, w_v, eidx_v, rows_a, rows_b, out_v,
              sem_a, sem_b, sem_self):
    w = _wid()
    b0 = w * BPW
    ngc = (C * K) // G

    pltpu.sync_copy(nid_hbm.at[pl.ds(w * (BPW * K // G), BPW * K // G)],
                    idx_v)
    pltpu.sync_copy(w_hbm.at[pl.ds(b0 * K, BPW * K)], w_v)
    pltpu.sync_copy(eid_hbm.at[pl.ds(w * (BPW // G), BPW // G)], eidx_v)

    self_cps = [
        _gather_rows(tab_hbm, eidx_v.at[g],
                     out_v.at[pl.ds(g * G, G)], sem_self)
        for g in range(BPW // G)
    ]
    for cp in self_cps:
        cp.wait()

    def fire(ci, dst, sem):
        for g in range(ngc):
            _gather_rows(tab_hbm, idx_v.at[ci * ngc + g],
                         dst.at[pl.ds(g * G, G)], sem)

    def drain(dst, sem):
        pltpu.make_async_copy(tab_hbm.at[pl.ds(0, C * K)], dst, sem).wait()

    def compute(ci, rows_v):
        for b in range(C):
            bb = ci * C + b
            a0 = jnp.zeros((L,), jnp.float32)
            a1 = jnp.zeros((L,), jnp.float32)
            tot = jnp.zeros((L,), jnp.float32)
            for k in range(K):
                ek = _vgather(w_v, jnp.full((L,), bb * K + k, jnp.int32))
                tot = tot + ek
                r = b * K + k
                a0 = a0 + ek * rows_v[r, 0:L]
                a1 = a1 + ek * rows_v[r, L:D]
            inv = jnp.full((L,), 1.0, jnp.float32) / tot
            out_v[bb, 0:L] = jnp.maximum(out_v[bb, 0:L] + a0 * inv, 0.0)
            out_v[bb, L:D] = jnp.maximum(out_v[bb, L:D] + a1 * inv, 0.0)

    fire(0, rows_a, sem_a)

    def pair(g2, carry):
        ci = g2 * 2
        fire(ci + 1, rows_b, sem_b)
        drain(rows_a, sem_a)
        compute(ci, rows_a)

        @pl.when(ci + 2 < NCHUNK)
        def _():
            fire(ci + 2, rows_a, sem_a)
        drain(rows_b, sem_b)
        compute(ci + 1, rows_b)
        return carry

    lax.fori_loop(0, NCHUNK // 2, pair, 0)
    pltpu.sync_copy(out_v, out_hbm.at[pl.ds(b0, BPW)])


@functools.cache
def _agg_sc():
  return pl.kernel(
    _agg_body,
    out_type=jax.ShapeDtypeStruct((B, D), jnp.float32),
    mesh=plsc.VectorSubcoreMesh(core_axis_name="c", subcore_axis_name="s",
                                num_cores=NC, num_subcores=NS),
    compiler_params=pltpu.CompilerParams(needs_layout_passes=False,
                                         use_tc_tiling_on_sc=False),
    scratch_types=[
        pltpu.VMEM((BPW * K // G, G), jnp.int32),
        pltpu.VMEM((BPW * K,), jnp.float32),
        pltpu.VMEM((BPW // G, G), jnp.int32),
        pltpu.VMEM((C * K, D), jnp.float32),
        pltpu.VMEM((C * K, D), jnp.float32),
        pltpu.VMEM((BPW, D), jnp.float32),
        pltpu.SemaphoreType.DMA,
        pltpu.SemaphoreType.DMA,
        pltpu.SemaphoreType.DMA,
    ],
  )


def kernel(user_emb, entity_ids, neigh_ent_ids, neigh_rel_ids,
           entity_table, relation_table, W):
    s = _scores_tc(user_emb.astype(jnp.float32), W.astype(jnp.float32),
                   relation_table.astype(jnp.float32))
    nid = _sigma(neigh_ent_ids.astype(jnp.int32)).reshape(B * K // G, G)
    eid = _sigma(entity_ids.astype(jnp.int32)).reshape(B // G, G)
    e = jnp.take_along_axis(s, neigh_rel_ids.astype(jnp.int32), axis=1,
                            mode="clip")
    tab_rm = _table_rowmajor_tc(entity_table.astype(jnp.float32).T)
    return _agg_sc()(nid, eid, e.reshape(B * K),
                     tab_rm.reshape(NEP, D))
